# baseline (speedup 1.0000x reference)
import jax
import jax.numpy as jnp
from jax.experimental import pallas as pl
from jax.experimental.pallas import tpu as pltpu

_P_PAD = 1024
_PAD_VAL = 1e18


def _block_kernel(rel_ref, x_ref, praw_ref, pout_ref, out_ref,
                  pt_scr, pn_scr, po_scr):
    bB = x_ref.shape[0]
    P, D = praw_ref.shape
    dn = (((1,), (0,)), ((), ()))
    hi = jax.lax.Precision.HIGHEST
    w = rel_ref[...] * rel_ref[...]

    @pl.when(pl.program_id(0) == 0)
    def _stage():
        pT = praw_ref[...].T
        pad = jnp.full((D, _P_PAD - P), _PAD_VAL, jnp.float32)
        pt = jnp.concatenate([pT, pad], axis=1)
        pt_scr[...] = pt
        pn_scr[...] = jax.lax.dot_general(
            w, pt * pt, dn, precision=hi,
            preferred_element_type=jnp.float32)
        poT = pout_ref[...].T
        po_scr[...] = jnp.concatenate(
            [poT, jnp.zeros((1, _P_PAD - P), jnp.float32)], axis=1)

    xb = x_ref[...]
    s = jax.lax.dot_general(xb * (-2.0 * w), pt_scr[...], dn, precision=hi,
                            preferred_element_type=jnp.float32)
    dist = pn_scr[...] + s

    m = jnp.min(dist, axis=1, keepdims=True)
    iota = jax.lax.broadcasted_iota(jnp.int32, (bB, _P_PAD), 1)
    cand = jnp.where(dist == m, iota, jnp.int32(2**30))
    j = jnp.min(cand, axis=1, keepdims=True)
    sel = jnp.where(iota == j, po_scr[...], 0.0)
    out_ref[...] = jnp.sum(sel, axis=1, keepdims=True)


def kernel(x, prototypes, prototype_outputs, relevance):
    B, D = x.shape
    bB = 512

    out = pl.pallas_call(
        _block_kernel,
        grid=(B // bB,),
        in_specs=[
            pl.BlockSpec((1, D), lambda i: (0, 0)),
            pl.BlockSpec((bB, D), lambda i: (i, 0)),
            pl.BlockSpec(prototypes.shape, lambda i: (0, 0)),
            pl.BlockSpec(prototype_outputs.shape, lambda i: (0, 0)),
        ],
        out_specs=pl.BlockSpec((bB, 1), lambda i: (i, 0)),
        out_shape=jax.ShapeDtypeStruct((B, 1), jnp.float32),
        scratch_shapes=[
            pltpu.VMEM((D, _P_PAD), jnp.float32),
            pltpu.VMEM((1, _P_PAD), jnp.float32),
            pltpu.VMEM((1, _P_PAD), jnp.float32),
        ],
    )(relevance.reshape(1, D), x, prototypes, prototype_outputs)
    return out


# speedup vs baseline: 1.0132x; 1.0132x over previous
import jax
import jax.numpy as jnp
from jax.experimental import pallas as pl
from jax.experimental.pallas import tpu as pltpu

_P_PAD = 1024
_PAD_VAL = 1e18


def _block_kernel(rel_ref, x_ref, praw_ref, pout_ref, out_ref,
                  pt_scr, pn_scr, po_scr):
    bB = x_ref.shape[0]
    P, D = praw_ref.shape
    dn = (((1,), (0,)), ((), ()))
    hi = jax.lax.Precision.HIGHEST
    w = rel_ref[...] * rel_ref[...]

    @pl.when(pl.program_id(0) == 0)
    def _stage():
        pT = praw_ref[...].T
        pad = jnp.full((D, _P_PAD - P), _PAD_VAL, jnp.float32)
        pt = jnp.concatenate([pT, pad], axis=1)
        pt_scr[...] = pt
        pn_scr[...] = jax.lax.dot_general(
            w, pt * pt, dn, precision=hi,
            preferred_element_type=jnp.float32)
        poT = pout_ref[...].T
        po_scr[...] = jnp.concatenate(
            [poT, jnp.zeros((1, _P_PAD - P), jnp.float32)], axis=1)

    xb = x_ref[...]
    s = jax.lax.dot_general(xb * (-2.0 * w), pt_scr[...], dn, precision=hi,
                            preferred_element_type=jnp.float32)
    dist = pn_scr[...] + s

    m = jnp.min(dist, axis=1, keepdims=True)
    iota = jax.lax.broadcasted_iota(jnp.int32, (bB, _P_PAD), 1)
    cand = jnp.where(dist == m, iota, jnp.int32(2**30))
    j = jnp.min(cand, axis=1, keepdims=True)
    sel = jnp.where(iota == j, po_scr[...], 0.0)
    out_ref[...] = jnp.sum(sel, axis=1, keepdims=True)


def kernel(x, prototypes, prototype_outputs, relevance):
    B, D = x.shape
    bB = 2048

    out = pl.pallas_call(
        _block_kernel,
        grid=(B // bB,),
        in_specs=[
            pl.BlockSpec((1, D), lambda i: (0, 0)),
            pl.BlockSpec((bB, D), lambda i: (i, 0)),
            pl.BlockSpec(prototypes.shape, lambda i: (0, 0)),
            pl.BlockSpec(prototype_outputs.shape, lambda i: (0, 0)),
        ],
        out_specs=pl.BlockSpec((bB, 1), lambda i: (i, 0)),
        out_shape=jax.ShapeDtypeStruct((B, 1), jnp.float32),
        scratch_shapes=[
            pltpu.VMEM((D, _P_PAD), jnp.float32),
            pltpu.VMEM((1, _P_PAD), jnp.float32),
            pltpu.VMEM((1, _P_PAD), jnp.float32),
        ],
    )(relevance.reshape(1, D), x, prototypes, prototype_outputs)
    return out


# speedup vs baseline: 1.5338x; 1.5138x over previous
import jax
import jax.numpy as jnp
from jax.experimental import pallas as pl
from jax.experimental.pallas import tpu as pltpu

_P_PAD = 1024
_PAD_VAL = 1e18


def _split3(v):
    c1 = v.astype(jnp.bfloat16)
    r1 = v - c1.astype(jnp.float32)
    c2 = r1.astype(jnp.bfloat16)
    c3 = (r1 - c2.astype(jnp.float32)).astype(jnp.bfloat16)
    return c1, c2, c3


def _block_kernel(rel_ref, x_ref, praw_ref, pout_ref, out_ref,
                  q_scr, pn_scr, po_scr):
    bB = x_ref.shape[0]
    P, D = praw_ref.shape
    dn = (((1,), (0,)), ((), ()))
    hi = jax.lax.Precision.HIGHEST
    w = rel_ref[...] * rel_ref[...]

    @pl.when(pl.program_id(0) == 0)
    def _stage():
        pT = praw_ref[...].T
        pad = jnp.full((D, _P_PAD - P), _PAD_VAL, jnp.float32)
        pt = jnp.concatenate([pT, pad], axis=1)
        pn_scr[...] = jax.lax.dot_general(
            w, pt * pt, dn, precision=hi,
            preferred_element_type=jnp.float32)
        wcol = rel_ref[...].T * rel_ref[...].T
        q1, q2, q3 = _split3(pt * (-2.0 * wcol))
        q_scr[...] = jnp.concatenate([q1, q2, q3, q1, q2, q1], axis=0)
        poT = pout_ref[...].T
        po_scr[...] = jnp.concatenate(
            [poT, jnp.zeros((1, _P_PAD - P), jnp.float32)], axis=1)

    xb = x_ref[...]
    x1, x2, x3 = _split3(xb)
    xa = jnp.concatenate([x1, x1, x1, x2, x2, x3], axis=1)
    s = jax.lax.dot_general(xa, q_scr[...], dn,
                            preferred_element_type=jnp.float32)
    dist = pn_scr[...] + s

    m = jnp.min(dist, axis=1, keepdims=True)
    iota = jax.lax.broadcasted_iota(jnp.int32, (bB, _P_PAD), 1)
    cand = jnp.where(dist == m, iota, jnp.int32(2**30))
    j = jnp.min(cand, axis=1, keepdims=True)
    sel = jnp.where(iota == j, po_scr[...], 0.0)
    out_ref[...] = jnp.sum(sel, axis=1, keepdims=True)


def kernel(x, prototypes, prototype_outputs, relevance):
    B, D = x.shape
    bB = 1024

    out = pl.pallas_call(
        _block_kernel,
        grid=(B // bB,),
        in_specs=[
            pl.BlockSpec((1, D), lambda i: (0, 0)),
            pl.BlockSpec((bB, D), lambda i: (i, 0)),
            pl.BlockSpec(prototypes.shape, lambda i: (0, 0)),
            pl.BlockSpec(prototype_outputs.shape, lambda i: (0, 0)),
        ],
        out_specs=pl.BlockSpec((bB, 1), lambda i: (i, 0)),
        out_shape=jax.ShapeDtypeStruct((B, 1), jnp.float32),
        scratch_shapes=[
            pltpu.VMEM((6 * D, _P_PAD), jnp.bfloat16),
            pltpu.VMEM((1, _P_PAD), jnp.float32),
            pltpu.VMEM((1, _P_PAD), jnp.float32),
        ],
    )(relevance.reshape(1, D), x, prototypes, prototype_outputs)
    return out


# speedup vs baseline: 1.6256x; 1.0599x over previous
import jax
import jax.numpy as jnp
from jax.experimental import pallas as pl
from jax.experimental.pallas import tpu as pltpu

_P_PAD = 1024
_PAD_VAL = 1e18


def _split3(v):
    c1 = v.astype(jnp.bfloat16)
    r1 = v - c1.astype(jnp.float32)
    c2 = r1.astype(jnp.bfloat16)
    c3 = (r1 - c2.astype(jnp.float32)).astype(jnp.bfloat16)
    return c1, c2, c3


def _block_kernel(rel_ref, x_ref, praw_ref, pout_ref, out_ref,
                  q_scr, pn_scr, po_scr, io_scr):
    bB = x_ref.shape[0]
    P, D = praw_ref.shape
    dn = (((1,), (0,)), ((), ()))
    hi = jax.lax.Precision.HIGHEST
    w = rel_ref[...] * rel_ref[...]

    @pl.when(pl.program_id(0) == 0)
    def _stage():
        pT = praw_ref[...].T
        pad = jnp.full((D, _P_PAD - P), _PAD_VAL, jnp.float32)
        pt = jnp.concatenate([pT, pad], axis=1)
        pn_scr[...] = jax.lax.dot_general(
            w, pt * pt, dn, precision=hi,
            preferred_element_type=jnp.float32)
        wcol = rel_ref[...].T * rel_ref[...].T
        q1, q2, q3 = _split3(pt * (-2.0 * wcol))
        q_scr[...] = jnp.concatenate([q1, q2, q3, q1, q2, q1], axis=0)
        poT = pout_ref[...].T
        po_scr[...] = jnp.concatenate(
            [poT, jnp.zeros((1, _P_PAD - P), jnp.float32)], axis=1)
        io_scr[...] = jax.lax.broadcasted_iota(
            jnp.int32, (1, _P_PAD), 1).astype(jnp.float32)

    xb = x_ref[...]
    x1, x2, x3 = _split3(xb)
    xa = jnp.concatenate([x1, x1, x1, x2, x2, x3], axis=1)
    s = jax.lax.dot_general(xa, q_scr[...], dn,
                            preferred_element_type=jnp.float32)
    dist = pn_scr[...] + s

    m = jnp.min(dist, axis=1, keepdims=True)
    iota = io_scr[...]
    cand = jnp.where(dist == m, iota, jnp.float32(2.0 ** 30))
    j = jnp.min(cand, axis=1, keepdims=True)
    sel = jnp.where(iota == j, po_scr[...], 0.0)
    out_ref[...] = jnp.sum(sel, axis=1, keepdims=True)


def kernel(x, prototypes, prototype_outputs, relevance):
    B, D = x.shape
    bB = 1024

    out = pl.pallas_call(
        _block_kernel,
        grid=(B // bB,),
        in_specs=[
            pl.BlockSpec((1, D), lambda i: (0, 0)),
            pl.BlockSpec((bB, D), lambda i: (i, 0)),
            pl.BlockSpec(prototypes.shape, lambda i: (0, 0)),
            pl.BlockSpec(prototype_outputs.shape, lambda i: (0, 0)),
        ],
        out_specs=pl.BlockSpec((bB, 1), lambda i: (i, 0)),
        out_shape=jax.ShapeDtypeStruct((B, 1), jnp.float32),
        scratch_shapes=[
            pltpu.VMEM((6 * D, _P_PAD), jnp.bfloat16),
            pltpu.VMEM((1, _P_PAD), jnp.float32),
            pltpu.VMEM((1, _P_PAD), jnp.float32),
            pltpu.VMEM((1, _P_PAD), jnp.float32),
        ],
    )(relevance.reshape(1, D), x, prototypes, prototype_outputs)
    return out
